# baseline (speedup 1.0000x reference)
import functools

import jax
import jax.numpy as jnp
from jax import lax
from jax.experimental import pallas as pl
from jax.experimental.pallas import tpu as pltpu
from jax.experimental.pallas import tpu_sc as plsc

VOCAB = 100000
SEQ = 200
DIM = 128
BATCH = 4096
TOT = BATCH * SEQ
NC = 2
NS = 16
NW = NC * NS
PER_W = TOT // NW
K = 128
NCH = PER_W // K
LANES = 16
NBUF = 4
NSB = 2
LEAD = 2
BLK = 8
NBLK = NCH // BLK
RPI = 4
SPLIT = 64
PR = 192 + SPLIT
assert SPLIT % 8 == 0 and (K - SPLIT) % LANES == 0

_mesh = plsc.VectorSubcoreMesh(core_axis_name="c", subcore_axis_name="s")


@functools.partial(
    pl.kernel,
    mesh=_mesh,
    out_type=jax.ShapeDtypeStruct((TOT, DIM), jnp.float32),
    scratch_types=[
        pltpu.VMEM((2, BLK, K), jnp.int32),
        pltpu.VMEM((NBUF, K, DIM), jnp.float32),
        pltpu.VMEM_SHARED((NS, NSB, K - SPLIT, DIM), jnp.float32),
        pltpu.VMEM((PR * DIM,), jnp.float32),
        pltpu.VMEM((K - SPLIT,), jnp.int32),
    ] + [pltpu.SemaphoreType.DMA] * (2 * NBUF + 2 * NSB + 2),
)
def _emb_kernel(ids_hbm, table_hbm, pos2_hbm, posf_hbm, out_hbm,
                idx_v, buf, sbuf, pos_v, idn_v, *sems):
    sg = sems[:NBUF]
    so1 = sems[NBUF:2 * NBUF]
    so2 = sems[2 * NBUF:2 * NBUF + NSB]
    sp = sems[2 * NBUF + NSB:2 * NBUF + 2 * NSB]
    si = sems[2 * NBUF + 2 * NSB]
    sa = sems[2 * NBUF + 2 * NSB + 1]
    sid = lax.axis_index("s")
    wid = sid * NC + lax.axis_index("c")
    base = wid * PER_W
    cbase = wid * NCH
    pltpu.sync_copy(posf_hbm.at[pl.ds(0, PR * DIM)], pos_v)
    for i in range((K - SPLIT) // LANES):
        idn_v[pl.ds(i * LANES, LANES)] = lax.iota(jnp.int32, LANES) + i * LANES

    def start_ids(blk, s):
        pltpu.async_copy(ids_hbm.at[pl.ds(cbase + blk * BLK, BLK)], idx_v.at[s], si)

    def wait_ids():
        pltpu.make_async_copy(ids_hbm.at[pl.ds(cbase, BLK)], idx_v.at[0], si).wait()

    def start_gather(pc, b, s, r):
        pltpu.async_copy(table_hbm.at[idx_v.at[s, r]], buf.at[b], sg[b])

    def wait_gather(b):
        pltpu.make_async_copy(table_hbm.at[idx_v.at[0, 0]], buf.at[b], sg[b]).wait()

    def start_init(pc, e):
        prow = lax.rem(pc * K, SEQ)
        pltpu.async_copy(
            pos2_hbm.at[pl.ds(prow + SPLIT, K - SPLIT)], sbuf.at[sid, e], sp[e])

    def wait_init(e):
        pltpu.make_async_copy(
            pos2_hbm.at[pl.ds(0, K - SPLIT)], sbuf.at[sid, e], sp[e]).wait()

    def start_add(b, e):
        pltpu.async_copy(
            buf.at[b, pl.ds(SPLIT, K - SPLIT)], sbuf.at[sid, e].at[idn_v],
            sa, add=True)

    def wait_add(b, e):
        pltpu.make_async_copy(
            buf.at[b, pl.ds(SPLIT, K - SPLIT)], sbuf.at[sid, e], sa).wait()

    def start_out1(cc, b):
        pltpu.async_copy(
            buf.at[b, pl.ds(0, SPLIT)],
            out_hbm.at[pl.ds(base + cc * K, SPLIT)], so1[b])

    def wait_out1(cc, b):
        pltpu.make_async_copy(
            buf.at[b, pl.ds(0, SPLIT)],
            out_hbm.at[pl.ds(base + cc * K, SPLIT)], so1[b]).wait()

    def start_out2(cc, e):
        pltpu.async_copy(
            sbuf.at[sid, e],
            out_hbm.at[pl.ds(base + cc * K + SPLIT, K - SPLIT)], so2[e])

    def wait_out2(cc, e):
        pltpu.make_async_copy(
            sbuf.at[sid, e],
            out_hbm.at[pl.ds(base + cc * K + SPLIT, K - SPLIT)], so2[e]).wait()

    def add_tec(cc, b):
        poff = lax.rem(cc * K, SEQ) * DIM

        def rows(r0, off):
            for rr in range(RPI):
                for j in range(DIM // LANES):
                    v = pos_v[pl.ds(off + rr * DIM + j * LANES, LANES)]
                    plsc.addupdate(
                        buf.at[b, r0 * RPI + rr, pl.ds(j * LANES, LANES)], v)
            return off + RPI * DIM

        lax.fori_loop(0, SPLIT // RPI, rows, poff)

    def step(cc, i, par, first, last):
        b = i % NBUF
        b2 = (i + LEAD) % NBUF
        e = i % NSB
        e1 = (i + 1) % NSB
        wait_gather(b)
        wait_init(e)
        start_add(b, e)
        if (not last) or (i < BLK - LEAD):
            if (not first) or (i >= LEAD):
                wait_out1(cc - LEAD, b2)
            s = par if (i + LEAD) // BLK == 0 else 1 - par
            start_gather(cc + LEAD, b2, s, (i + LEAD) % BLK)
        add_tec(cc, b)
        wait_add(b, e)
        start_out2(cc, e)
        start_out1(cc, b)
        if (not first) or (i >= 1):
            wait_out2(cc - 1, e1)
        if (not last) or (i < BLK - 1):
            start_init(cc + 1, e1)

    def block_body(g, par, first=False, last=False):
        c0 = g * BLK
        if not last:
            start_ids(g + 1, 1 - par)
        for i in range(BLK):
            if i == BLK - LEAD and not last:
                wait_ids()
            step(c0 + i, i, par, first, last)

    start_ids(0, 0)
    wait_ids()
    for b in range(LEAD):
        start_gather(b, b, 0, b)
    start_init(0, 0)

    block_body(0, 0, first=True)

    def mid(g, _):
        block_body(g, lax.rem(g, 2))
        return 0

    lax.fori_loop(1, NBLK - 1, mid, 0)

    block_body(NBLK - 1, (NBLK - 1) % 2, last=True)
    wait_out2(NCH - 1, (NCH - 1) % NSB)
    for cc in range(NCH - NBUF, NCH):
        wait_out1(cc, cc % NBUF)


def kernel(input_ids, table, pos_embed):
    ids2d = input_ids.reshape(NCH * NW, K)
    pos2d = pos_embed.reshape(SEQ, DIM)
    pos2 = jnp.concatenate([pos2d, pos2d], axis=0)
    out = _emb_kernel(ids2d, table, pos2, pos2.reshape(-1))
    return out.reshape(BATCH, SEQ, DIM)


# speedup vs baseline: 1.0485x; 1.0485x over previous
import functools

import jax
import jax.numpy as jnp
from jax import lax
from jax.experimental import pallas as pl
from jax.experimental.pallas import tpu as pltpu
from jax.experimental.pallas import tpu_sc as plsc

VOCAB = 100000
SEQ = 200
DIM = 128
BATCH = 4096
TOT = BATCH * SEQ
NC = 2
NS = 16
NW = NC * NS
PER_W = TOT // NW
K = 128
NCH = PER_W // K
LANES = 16
NBUF = 4
NSB = 2
LEAD = 2
BLK = 8
NBLK = NCH // BLK
RPI = 4
DMA_POS = (3, 7)

_mesh = plsc.VectorSubcoreMesh(core_axis_name="c", subcore_axis_name="s")


@functools.partial(
    pl.kernel,
    mesh=_mesh,
    out_type=jax.ShapeDtypeStruct((TOT, DIM), jnp.float32),
    scratch_types=[
        pltpu.VMEM((2, BLK, K), jnp.int32),
        pltpu.VMEM((NBUF, K, DIM), jnp.float32),
        pltpu.VMEM_SHARED((NS, NSB, K, DIM), jnp.float32),
        pltpu.VMEM((SEQ * DIM,), jnp.float32),
        pltpu.VMEM((K,), jnp.int32),
    ] + [pltpu.SemaphoreType.DMA] * (2 * NBUF + 2 * NSB + 2),
)
def _emb_kernel(ids_hbm, table_hbm, pos2_hbm, posf_hbm, out_hbm,
                idx_v, buf, sbuf, pos_v, idn_v, *sems):
    sg = sems[:NBUF]
    so1 = sems[NBUF:2 * NBUF]
    so2 = sems[2 * NBUF:2 * NBUF + NSB]
    sp = sems[2 * NBUF + NSB:2 * NBUF + 2 * NSB]
    si = sems[2 * NBUF + 2 * NSB]
    sa = sems[2 * NBUF + 2 * NSB + 1]
    sid = lax.axis_index("s")
    wid = sid * NC + lax.axis_index("c")
    base = wid * PER_W
    cbase = wid * NCH
    pltpu.sync_copy(posf_hbm, pos_v)
    for i in range(K // LANES):
        idn_v[pl.ds(i * LANES, LANES)] = lax.iota(jnp.int32, LANES) + i * LANES

    def start_ids(blk, s):
        pltpu.async_copy(ids_hbm.at[pl.ds(cbase + blk * BLK, BLK)], idx_v.at[s], si)

    def wait_ids():
        pltpu.make_async_copy(ids_hbm.at[pl.ds(cbase, BLK)], idx_v.at[0], si).wait()

    def start_gather(pc, b, s, r):
        pltpu.async_copy(table_hbm.at[idx_v.at[s, r]], buf.at[b], sg[b])

    def wait_gather(b):
        pltpu.make_async_copy(table_hbm.at[idx_v.at[0, 0]], buf.at[b], sg[b]).wait()

    def start_init(pc, e):
        prow = lax.rem(pc * K, SEQ)
        pltpu.async_copy(pos2_hbm.at[pl.ds(prow, K)], sbuf.at[sid, e], sp[e])

    def wait_init(e):
        pltpu.make_async_copy(
            pos2_hbm.at[pl.ds(0, K)], sbuf.at[sid, e], sp[e]).wait()

    def start_add(b, e):
        pltpu.async_copy(buf.at[b], sbuf.at[sid, e].at[idn_v], sa, add=True)

    def wait_add(b, e):
        pltpu.make_async_copy(buf.at[b], sbuf.at[sid, e], sa).wait()

    def start_out1(cc, b):
        pltpu.async_copy(buf.at[b], out_hbm.at[pl.ds(base + cc * K, K)], so1[b])

    def wait_out1(cc, b):
        pltpu.make_async_copy(
            buf.at[b], out_hbm.at[pl.ds(base + cc * K, K)], so1[b]).wait()

    def start_out2(cc, e):
        pltpu.async_copy(sbuf.at[sid, e], out_hbm.at[pl.ds(base + cc * K, K)], so2[e])

    def wait_out2(cc, e):
        pltpu.make_async_copy(
            sbuf.at[sid, e], out_hbm.at[pl.ds(base + cc * K, K)], so2[e]).wait()

    def add_tec(cc, b):
        prow = lax.rem(cc * K, SEQ)
        n1 = lax.min(SEQ - prow, K)

        def rows(r0, off):
            for rr in range(RPI):
                for j in range(DIM // LANES):
                    v = pos_v[pl.ds(off + rr * DIM + j * LANES, LANES)]
                    plsc.addupdate(
                        buf.at[b, r0 * RPI + rr, pl.ds(j * LANES, LANES)], v)
            return off + RPI * DIM

        lax.fori_loop(0, n1 // RPI, rows, prow * DIM)
        lax.fori_loop(n1 // RPI, K // RPI, rows, 0)

    def step(cc, i, par, first, last):
        b = i % NBUF
        b2 = (i + LEAD) % NBUF
        if (not last) or (i < BLK - LEAD):
            if ((i - LEAD) % BLK) not in DMA_POS:
                if (not first) or (i >= LEAD):
                    wait_out1(cc - LEAD, b2)
            s = par if (i + LEAD) // BLK == 0 else 1 - par
            start_gather(cc + LEAD, b2, s, (i + LEAD) % BLK)
        wait_gather(b)
        if i in DMA_POS:
            e = DMA_POS.index(i)
            wait_init(e)
            start_add(b, e)
            wait_add(b, e)
            start_out2(cc, e)
        else:
            add_tec(cc, b)
            start_out1(cc, b)
        if i == 6 and not first:
            wait_out2(cc - 3, 0)
            if not last:
                start_init(cc + 5, 0)
        if i == 2 and not first:
            wait_out2(cc - 3, 1)
            start_init(cc + 5, 1)

    def block_body(g, par, first=False, last=False):
        c0 = g * BLK
        if not last:
            start_ids(g + 1, 1 - par)
        for i in range(BLK):
            if i == BLK - LEAD and not last:
                wait_ids()
            step(c0 + i, i, par, first, last)

    start_ids(0, 0)
    wait_ids()
    for b in range(LEAD):
        start_gather(b, b, 0, b)
    start_init(DMA_POS[0], 0)
    start_init(DMA_POS[1], 1)

    block_body(0, 0, first=True)
    wait_out2(DMA_POS[0], 0)
    start_init(BLK + DMA_POS[0], 0)

    def mid(g, _):
        block_body(g, lax.rem(g, 2))
        return 0

    lax.fori_loop(1, NBLK - 1, mid, 0)

    block_body(NBLK - 1, (NBLK - 1) % 2, last=True)
    c0 = (NBLK - 1) * BLK
    for i in (4, 5, 6):
        wait_out1(c0 + i, i % NBUF)
    wait_out2(c0 + 7, 1)


def kernel(input_ids, table, pos_embed):
    ids2d = input_ids.reshape(NCH * NW, K)
    pos2d = pos_embed.reshape(SEQ, DIM)
    pos2 = jnp.concatenate([pos2d, pos2d], axis=0)
    out = _emb_kernel(ids2d, table, pos2, pos_embed.reshape(SEQ * DIM))
    return out.reshape(BATCH, SEQ, DIM)
